# baseline (speedup 1.0000x reference)
import jax
import jax.numpy as jnp
from jax.experimental import pallas as pl
from jax.experimental.pallas import tpu as pltpu


def _mm_bf16_kernel(x_ref, w_ref, o_ref):
    o_ref[...] = jnp.dot(
        x_ref[...], w_ref[...], preferred_element_type=jnp.float32
    ).astype(jnp.bfloat16)


def _mm_q8_kernel(x_ref, w_ref, o_ref, sc_ref):
    y = jnp.dot(x_ref[...], w_ref[...], preferred_element_type=jnp.float32)
    s = jnp.maximum(jnp.max(jnp.abs(y)), 1e-30)
    o_ref[...] = jnp.clip(jnp.round(y * (127.0 / s)), -127.0, 127.0).astype(jnp.int8)
    sc_ref[...] = jnp.full(sc_ref.shape, s * (1.0 / 127.0), jnp.float32)


def _pass1_kernel(wb_ref, a_ref, s1_ref, b1_ref, u1_ref, aq_ref, asc_ref, macc_ref):
    v = pl.program_id(1)
    w = wb_ref[...]
    wv = jnp.where(v == 0, w[0, 0], jnp.where(v == 1, w[1, 0], w[2, 0]))
    contrib = wv * a_ref[0]

    @pl.when(v == 0)
    def _():
        macc_ref[...] = contrib

    @pl.when(v == 1)
    def _():
        macc_ref[...] += contrib

    @pl.when(v == 2)
    def _():
        m = macc_ref[...] + contrib
        u1_ref[...] = (
            jnp.dot(
                m.astype(jnp.bfloat16), s1_ref[...],
                preferred_element_type=jnp.float32,
            )
            + b1_ref[...]
        )
        s = jnp.maximum(jnp.max(jnp.abs(m)), 1e-30)
        aq_ref[0] = jnp.clip(jnp.round(m * (127.0 / s)), -127.0, 127.0).astype(
            jnp.int8
        )
        asc_ref[...] = jnp.full(asc_ref.shape, s * (1.0 / 127.0), jnp.float32)


def _pass2_kernel(aq_ref, asc_ref, s2_ref, s2sc_ref, u1_ref, b2_ref, o_ref):
    acc = jnp.dot(aq_ref[0], s2_ref[...], preferred_element_type=jnp.int32)
    scale = asc_ref[0, 0, 0] * s2sc_ref[0, 0]
    u2 = acc.astype(jnp.float32) * scale + b2_ref[...]
    o_ref[...] = (u2 + u1_ref[...]) * 0.5


def kernel(feature, A, W1, b1, W2, b2, weight_b):
    n, f = feature.shape
    out = W1.shape[1]
    bm = 200 if n % 200 == 0 else n
    gi = n // bm

    b1r = b1.reshape(1, out)
    b2r = b2.reshape(1, out)

    support1 = pl.pallas_call(
        _mm_bf16_kernel,
        out_shape=jax.ShapeDtypeStruct((n, out), jnp.bfloat16),
    )(feature, W1)

    u1, a_q, a_sc = pl.pallas_call(
        _pass1_kernel,
        grid=(gi, 3),
        in_specs=[
            pl.BlockSpec((3, 1), lambda i, v: (0, 0)),
            pl.BlockSpec((1, bm, n), lambda i, v: (v, i, 0)),
            pl.BlockSpec((n, out), lambda i, v: (0, 0)),
            pl.BlockSpec((1, out), lambda i, v: (0, 0)),
        ],
        out_specs=[
            pl.BlockSpec((bm, out), lambda i, v: (i, 0)),
            pl.BlockSpec((1, bm, n), lambda i, v: (i, 0, 0)),
            pl.BlockSpec((1, 1, 128), lambda i, v: (i, 0, 0)),
        ],
        out_shape=[
            jax.ShapeDtypeStruct((n, out), jnp.float32),
            jax.ShapeDtypeStruct((gi, bm, n), jnp.int8),
            jax.ShapeDtypeStruct((gi, 1, 128), jnp.float32),
        ],
        scratch_shapes=[pltpu.VMEM((bm, n), jnp.float32)],
        compiler_params=pltpu.CompilerParams(
            dimension_semantics=("parallel", "arbitrary"),
        ),
    )(weight_b, A, support1, b1r)

    support2, s2_sc = pl.pallas_call(
        _mm_q8_kernel,
        out_shape=[
            jax.ShapeDtypeStruct((n, out), jnp.int8),
            jax.ShapeDtypeStruct((1, 128), jnp.float32),
        ],
    )(u1, W2)

    result = pl.pallas_call(
        _pass2_kernel,
        grid=(gi,),
        in_specs=[
            pl.BlockSpec((1, bm, n), lambda i: (i, 0, 0)),
            pl.BlockSpec((1, 1, 128), lambda i: (i, 0, 0)),
            pl.BlockSpec((n, out), lambda i: (0, 0)),
            pl.BlockSpec((1, 128), lambda i: (0, 0)),
            pl.BlockSpec((bm, out), lambda i: (i, 0)),
            pl.BlockSpec((1, out), lambda i: (0, 0)),
        ],
        out_specs=pl.BlockSpec((bm, out), lambda i: (i, 0)),
        out_shape=jax.ShapeDtypeStruct((n, out), jnp.float32),
        compiler_params=pltpu.CompilerParams(
            dimension_semantics=("parallel",),
        ),
    )(a_q, a_sc, support2, s2_sc, u1, b2r)

    return result


# speedup vs baseline: 1.2789x; 1.2789x over previous
import jax
import jax.numpy as jnp
from jax.experimental import pallas as pl
from jax.experimental.pallas import tpu as pltpu


def _pass1_kernel(wb_ref, a_ref, feat_ref, w1_ref, b1_ref, u1_ref, abf_ref, s1_ref):
    i = pl.program_id(0)
    v = pl.program_id(1)

    @pl.when(jnp.logical_and(i == 0, v == 0))
    def _():
        s1_ref[...] = jnp.dot(
            feat_ref[...], w1_ref[...], preferred_element_type=jnp.float32
        ).astype(jnp.bfloat16)

    w = wb_ref[...]
    wv = jnp.where(v == 0, w[0, 0], jnp.where(v == 1, w[1, 0], w[2, 0]))
    contrib = (wv * a_ref[0]).astype(jnp.bfloat16)

    @pl.when(v == 0)
    def _():
        abf_ref[...] = contrib

    @pl.when(v > 0)
    def _():
        abf_ref[...] += contrib

    @pl.when(v == 2)
    def _():
        u1_ref[...] = (
            jnp.dot(abf_ref[...], s1_ref[...], preferred_element_type=jnp.float32)
            + b1_ref[...]
        )


def _pass2_kernel(abf_ref, u1full_ref, w2_ref, u1_ref, b2_ref, o_ref, s2_ref):
    i = pl.program_id(0)

    @pl.when(i == 0)
    def _():
        s2_ref[...] = jnp.dot(
            u1full_ref[...], w2_ref[...], preferred_element_type=jnp.float32
        ).astype(jnp.bfloat16)

    u2 = (
        jnp.dot(abf_ref[...], s2_ref[...], preferred_element_type=jnp.float32)
        + b2_ref[...]
    )
    o_ref[...] = (u2 + u1_ref[...]) * 0.5


def kernel(feature, A, W1, b1, W2, b2, weight_b):
    n, f = feature.shape
    out = W1.shape[1]
    bm = 200 if n % 200 == 0 else n
    gi = n // bm
    bm2 = 1000 if n % 1000 == 0 else n
    gi2 = n // bm2

    b1r = b1.reshape(1, out)
    b2r = b2.reshape(1, out)

    u1, a_bf = pl.pallas_call(
        _pass1_kernel,
        grid=(gi, 3),
        in_specs=[
            pl.BlockSpec((3, 1), lambda i, v: (0, 0)),
            pl.BlockSpec((1, bm, n), lambda i, v: (v, i, 0)),
            pl.BlockSpec((n, f), lambda i, v: (0, 0)),
            pl.BlockSpec((f, out), lambda i, v: (0, 0)),
            pl.BlockSpec((1, out), lambda i, v: (0, 0)),
        ],
        out_specs=[
            pl.BlockSpec((bm, out), lambda i, v: (i, 0)),
            pl.BlockSpec((bm, n), lambda i, v: (i, 0)),
        ],
        out_shape=[
            jax.ShapeDtypeStruct((n, out), jnp.float32),
            jax.ShapeDtypeStruct((n, n), jnp.bfloat16),
        ],
        scratch_shapes=[pltpu.VMEM((n, out), jnp.bfloat16)],
        compiler_params=pltpu.CompilerParams(
            dimension_semantics=("arbitrary", "arbitrary"),
        ),
    )(weight_b, A, feature, W1, b1r)

    result = pl.pallas_call(
        _pass2_kernel,
        grid=(gi2,),
        in_specs=[
            pl.BlockSpec((bm2, n), lambda i: (i, 0)),
            pl.BlockSpec((n, out), lambda i: (0, 0)),
            pl.BlockSpec((out, out), lambda i: (0, 0)),
            pl.BlockSpec((bm2, out), lambda i: (i, 0)),
            pl.BlockSpec((1, out), lambda i: (0, 0)),
        ],
        out_specs=pl.BlockSpec((bm2, out), lambda i: (i, 0)),
        out_shape=jax.ShapeDtypeStruct((n, out), jnp.float32),
        scratch_shapes=[pltpu.VMEM((n, out), jnp.bfloat16)],
        compiler_params=pltpu.CompilerParams(
            dimension_semantics=("arbitrary",),
        ),
    )(a_bf, u1, W2, u1, b2r)

    return result


# speedup vs baseline: 1.4417x; 1.1273x over previous
import jax
import jax.numpy as jnp
from jax.experimental import pallas as pl
from jax.experimental.pallas import tpu as pltpu


def _pass1_kernel(wb_ref, a_ref, feat_ref, w1_ref, b1_ref, u1_ref, abf_ref, s1_ref):
    i = pl.program_id(0)

    @pl.when(i == 0)
    def _():
        s1_ref[...] = jnp.dot(
            feat_ref[...], w1_ref[...], preferred_element_type=jnp.float32
        ).astype(jnp.bfloat16)

    w = wb_ref[...]
    m = (
        w[0, 0] * a_ref[0] + w[1, 0] * a_ref[1] + w[2, 0] * a_ref[2]
    ).astype(jnp.bfloat16)
    abf_ref[...] = m
    u1_ref[...] = (
        jnp.dot(m, s1_ref[...], preferred_element_type=jnp.float32)
        + b1_ref[...]
    )


def _pass2_kernel(abf_ref, u1full_ref, w2_ref, u1_ref, b2_ref, o_ref, s2_ref):
    i = pl.program_id(0)

    @pl.when(i == 0)
    def _():
        s2_ref[...] = jnp.dot(
            u1full_ref[...], w2_ref[...], preferred_element_type=jnp.float32
        ).astype(jnp.bfloat16)

    u2 = (
        jnp.dot(abf_ref[...], s2_ref[...], preferred_element_type=jnp.float32)
        + b2_ref[...]
    )
    o_ref[...] = (u2 + u1_ref[...]) * 0.5


def kernel(feature, A, W1, b1, W2, b2, weight_b):
    n, f = feature.shape
    out = W1.shape[1]
    bm = 80 if n % 80 == 0 else n
    gi = n // bm
    bm2 = 1000 if n % 1000 == 0 else n
    gi2 = n // bm2

    b1r = b1.reshape(1, out)
    b2r = b2.reshape(1, out)

    u1, a_bf = pl.pallas_call(
        _pass1_kernel,
        grid=(gi,),
        in_specs=[
            pl.BlockSpec((3, 1), lambda i: (0, 0)),
            pl.BlockSpec((3, bm, n), lambda i: (0, i, 0)),
            pl.BlockSpec((n, f), lambda i: (0, 0)),
            pl.BlockSpec((f, out), lambda i: (0, 0)),
            pl.BlockSpec((1, out), lambda i: (0, 0)),
        ],
        out_specs=[
            pl.BlockSpec((bm, out), lambda i: (i, 0)),
            pl.BlockSpec((bm, n), lambda i: (i, 0)),
        ],
        out_shape=[
            jax.ShapeDtypeStruct((n, out), jnp.float32),
            jax.ShapeDtypeStruct((n, n), jnp.bfloat16),
        ],
        scratch_shapes=[pltpu.VMEM((n, out), jnp.bfloat16)],
        compiler_params=pltpu.CompilerParams(
            dimension_semantics=("arbitrary",),
        ),
    )(weight_b, A, feature, W1, b1r)

    result = pl.pallas_call(
        _pass2_kernel,
        grid=(gi2,),
        in_specs=[
            pl.BlockSpec((bm2, n), lambda i: (i, 0)),
            pl.BlockSpec((n, out), lambda i: (0, 0)),
            pl.BlockSpec((out, out), lambda i: (0, 0)),
            pl.BlockSpec((bm2, out), lambda i: (i, 0)),
            pl.BlockSpec((1, out), lambda i: (0, 0)),
        ],
        out_specs=pl.BlockSpec((bm2, out), lambda i: (i, 0)),
        out_shape=jax.ShapeDtypeStruct((n, out), jnp.float32),
        scratch_shapes=[pltpu.VMEM((n, out), jnp.bfloat16)],
        compiler_params=pltpu.CompilerParams(
            dimension_semantics=("arbitrary",),
        ),
    )(a_bf, u1, W2, u1, b2r)

    return result
